# baseline (speedup 1.0000x reference)
import functools

import jax
import jax.numpy as jnp
from jax import lax
from jax.experimental import pallas as pl
from jax.experimental.pallas import tpu as pltpu
from jax.experimental.pallas import tpu_sc as plsc

ACTION_SIZE = 4
POINT_SIZE = 11
EMBED_DIM = 128
LANES = 16
NUM_WORKERS = 32
BATCH = 16384
ROWS_PER_W = BATCH // NUM_WORKERS
CHUNKS_PER_W = ROWS_PER_W // LANES
BLOCK_B = 8192


def _tc_sims_body(ew_ref, av_ref, out_ref):
    ew = ew_ref[...]
    av = av_ref[0]
    num = jax.lax.dot_general(
        av, ew, (((0,), (1,)), ((), ())),
        precision=jax.lax.Precision.HIGHEST,
        preferred_element_type=jnp.float32)
    n1 = jnp.sqrt(jnp.sum(ew * ew, axis=1, keepdims=True)).T
    n2 = jnp.sqrt(jnp.sum(av * av, axis=0, keepdims=True)).T
    out_ref[...] = num / jnp.maximum(n1 * n2, 1e-8)


def _sims_transposed(embedded_words, action_vectors):
    return pl.pallas_call(
        _tc_sims_body,
        grid=(BATCH // BLOCK_B,),
        in_specs=[
            pl.BlockSpec((BLOCK_B, EMBED_DIM), lambda i: (i, 0)),
            pl.BlockSpec((1, EMBED_DIM, POINT_SIZE), lambda i: (0, 0, 0)),
        ],
        out_specs=pl.BlockSpec((POINT_SIZE, BLOCK_B), lambda i: (0, i)),
        out_shape=jax.ShapeDtypeStruct((POINT_SIZE, BATCH), jnp.float32),
    )(embedded_words, action_vectors)


@functools.partial(
    pl.kernel,
    mesh=plsc.VectorSubcoreMesh(core_axis_name="c", subcore_axis_name="s"),
    out_type=jax.ShapeDtypeStruct((BATCH, ACTION_SIZE), jnp.float32),
    scratch_types=[
        pltpu.VMEM((POINT_SIZE, ROWS_PER_W), jnp.float32),
        pltpu.VMEM((ROWS_PER_W, ACTION_SIZE), jnp.float32),
    ],
    compiler_params=pltpu.CompilerParams(needs_layout_passes=False),
)
def _sc_onehot(sims_hbm, out_hbm, sims_v, oht_v):
    wid = lax.axis_index("s") * 2 + lax.axis_index("c")
    base = wid * ROWS_PER_W
    pltpu.sync_copy(sims_hbm.at[:, pl.ds(base, ROWS_PER_W)], sims_v)
    one = jnp.full((LANES,), 1.0, jnp.float32)
    zero = jnp.zeros((LANES,), jnp.float32)
    lane = lax.iota(jnp.int32, LANES)

    @plsc.parallel_loop(0, CHUNKS_PER_W, 1, unroll=4)
    def chunk(i):
        s = [sims_v[p, pl.ds(i * LANES, LANES)] for p in range(POINT_SIZE)]
        p0 = jnp.maximum(jnp.maximum(s[0], s[1]), jnp.maximum(s[2], s[3]))
        p1 = jnp.maximum(jnp.maximum(jnp.maximum(s[4], s[5]),
                                     jnp.maximum(s[6], s[7])), s[8])
        p2 = s[9]
        p3 = s[10]
        best = jnp.maximum(jnp.maximum(p0, p1), jnp.maximum(p2, p3))
        w0 = p0 >= best
        w1 = (p1 >= best) & (~w0)
        w2 = (p2 >= best) & (~(w0 | w1))
        w3 = ~(w0 | w1 | w2)
        row_idx = lane + i * LANES
        for a, w in enumerate((w0, w1, w2, w3)):
            col_idx = jnp.full((LANES,), a, jnp.int32)
            plsc.store_scatter(oht_v, [row_idx, col_idx],
                               jnp.where(w, one, zero))

    pltpu.sync_copy(oht_v, out_hbm.at[pl.ds(base, ROWS_PER_W)])


def kernel(embedded_words, action_vectors):
    return _sc_onehot(_sims_transposed(embedded_words, action_vectors))


# speedup vs baseline: 1.0097x; 1.0097x over previous
import functools

import jax
import jax.numpy as jnp
from jax import lax
from jax.experimental import pallas as pl
from jax.experimental.pallas import tpu as pltpu
from jax.experimental.pallas import tpu_sc as plsc

ACTION_SIZE = 4
POINT_SIZE = 11
EMBED_DIM = 128
LANES = 16
NUM_WORKERS = 32
BATCH = 16384
ROWS_PER_W = BATCH // NUM_WORKERS
CHUNKS_PER_W = ROWS_PER_W // LANES
BLOCK_B = 4096


def _tc_sims_body(ew_ref, av_ref, out_ref):
    ew = ew_ref[...]
    av = av_ref[0]
    num = jax.lax.dot_general(
        av, ew, (((0,), (1,)), ((), ())),
        precision=jax.lax.Precision.HIGHEST,
        preferred_element_type=jnp.float32)
    n2sq = jnp.sum(av * av, axis=0, keepdims=True).T
    out_ref[...] = num * jax.lax.rsqrt(n2sq)


def _sims_transposed(embedded_words, action_vectors):
    return pl.pallas_call(
        _tc_sims_body,
        grid=(BATCH // BLOCK_B,),
        in_specs=[
            pl.BlockSpec((BLOCK_B, EMBED_DIM), lambda i: (i, 0)),
            pl.BlockSpec((1, EMBED_DIM, POINT_SIZE), lambda i: (0, 0, 0)),
        ],
        out_specs=pl.BlockSpec((POINT_SIZE, BLOCK_B), lambda i: (0, i)),
        out_shape=jax.ShapeDtypeStruct((POINT_SIZE, BATCH), jnp.float32),
    )(embedded_words, action_vectors)


@functools.partial(
    pl.kernel,
    mesh=plsc.VectorSubcoreMesh(core_axis_name="c", subcore_axis_name="s"),
    out_type=jax.ShapeDtypeStruct((BATCH, ACTION_SIZE), jnp.float32),
    scratch_types=[
        pltpu.VMEM((POINT_SIZE, ROWS_PER_W), jnp.float32),
        pltpu.VMEM((ROWS_PER_W, ACTION_SIZE), jnp.float32),
    ],
    compiler_params=pltpu.CompilerParams(needs_layout_passes=False),
)
def _sc_onehot(sims_hbm, out_hbm, sims_v, oht_v):
    wid = lax.axis_index("s") * 2 + lax.axis_index("c")
    base = wid * ROWS_PER_W
    pltpu.sync_copy(sims_hbm.at[:, pl.ds(base, ROWS_PER_W)], sims_v)
    one = jnp.full((LANES,), 1.0, jnp.float32)
    zero = jnp.zeros((LANES,), jnp.float32)
    lane = lax.iota(jnp.int32, LANES)

    @plsc.parallel_loop(0, CHUNKS_PER_W, 1, unroll=4)
    def chunk(i):
        s = [sims_v[p, pl.ds(i * LANES, LANES)] for p in range(POINT_SIZE)]
        p0 = jnp.maximum(jnp.maximum(s[0], s[1]), jnp.maximum(s[2], s[3]))
        p1 = jnp.maximum(jnp.maximum(jnp.maximum(s[4], s[5]),
                                     jnp.maximum(s[6], s[7])), s[8])
        p2 = s[9]
        p3 = s[10]
        best = jnp.maximum(jnp.maximum(p0, p1), jnp.maximum(p2, p3))
        w0 = p0 >= best
        w1 = (p1 >= best) & (~w0)
        w2 = (p2 >= best) & (~(w0 | w1))
        w3 = ~(w0 | w1 | w2)
        row_idx = lane + i * LANES
        for a, w in enumerate((w0, w1, w2, w3)):
            col_idx = jnp.full((LANES,), a, jnp.int32)
            plsc.store_scatter(oht_v, [row_idx, col_idx],
                               jnp.where(w, one, zero))

    pltpu.sync_copy(oht_v, out_hbm.at[pl.ds(base, ROWS_PER_W)])


def kernel(embedded_words, action_vectors):
    return _sc_onehot(_sims_transposed(embedded_words, action_vectors))


# speedup vs baseline: 1.0104x; 1.0007x over previous
import functools

import jax
import jax.numpy as jnp
from jax import lax
from jax.experimental import pallas as pl
from jax.experimental.pallas import tpu as pltpu
from jax.experimental.pallas import tpu_sc as plsc

ACTION_SIZE = 4
POINT_SIZE = 11
EMBED_DIM = 128
LANES = 16
NUM_WORKERS = 32
BATCH = 16384
ROWS_PER_W = BATCH // NUM_WORKERS
CHUNKS_PER_W = ROWS_PER_W // LANES
BLOCK_B = 4096


def _tc_sims_body(ew_ref, av_ref, out_ref):
    ew = ew_ref[...]
    av = av_ref[0]
    num = jax.lax.dot_general(
        av, ew, (((0,), (1,)), ((), ())),
        precision=jax.lax.Precision.HIGHEST,
        preferred_element_type=jnp.float32)
    n2sq = jnp.sum(av * av, axis=0, keepdims=True).T
    out_ref[...] = num * jax.lax.rsqrt(n2sq)


def _sims_transposed(embedded_words, action_vectors):
    return pl.pallas_call(
        _tc_sims_body,
        grid=(BATCH // BLOCK_B,),
        in_specs=[
            pl.BlockSpec((BLOCK_B, EMBED_DIM), lambda i: (i, 0)),
            pl.BlockSpec((1, EMBED_DIM, POINT_SIZE), lambda i: (0, 0, 0)),
        ],
        out_specs=pl.BlockSpec((POINT_SIZE, BLOCK_B), lambda i: (0, i)),
        out_shape=jax.ShapeDtypeStruct((POINT_SIZE, BATCH), jnp.float32),
    )(embedded_words, action_vectors)


@functools.partial(
    pl.kernel,
    mesh=plsc.VectorSubcoreMesh(core_axis_name="c", subcore_axis_name="s"),
    out_type=jax.ShapeDtypeStruct((BATCH, ACTION_SIZE), jnp.float32),
    scratch_types=[
        pltpu.VMEM((POINT_SIZE, ROWS_PER_W), jnp.float32),
        pltpu.VMEM((ROWS_PER_W, ACTION_SIZE), jnp.float32),
        pltpu.SemaphoreType.DMA,
        pltpu.SemaphoreType.DMA,
    ],
    compiler_params=pltpu.CompilerParams(needs_layout_passes=False),
)
def _sc_onehot(sims_hbm, out_hbm, sims_v, oht_v, sem0, sem1):
    wid = lax.axis_index("s") * 2 + lax.axis_index("c")
    base = wid * ROWS_PER_W
    half = ROWS_PER_W // 2
    c0 = pltpu.async_copy(sims_hbm.at[:, pl.ds(base, half)],
                          sims_v.at[:, pl.ds(0, half)], sem0)
    c1 = pltpu.async_copy(sims_hbm.at[:, pl.ds(base + half, half)],
                          sims_v.at[:, pl.ds(half, half)], sem1)
    one = jnp.full((LANES,), 1.0, jnp.float32)
    zero = jnp.zeros((LANES,), jnp.float32)
    lane = lax.iota(jnp.int32, LANES)

    def chunk(i):
        s = [sims_v[p, pl.ds(i * LANES, LANES)] for p in range(POINT_SIZE)]
        p0 = jnp.maximum(jnp.maximum(s[0], s[1]), jnp.maximum(s[2], s[3]))
        p1 = jnp.maximum(jnp.maximum(jnp.maximum(s[4], s[5]),
                                     jnp.maximum(s[6], s[7])), s[8])
        p2 = s[9]
        p3 = s[10]
        best = jnp.maximum(jnp.maximum(p0, p1), jnp.maximum(p2, p3))
        w0 = p0 >= best
        w1 = (p1 >= best) & (~w0)
        w2 = (p2 >= best) & (~(w0 | w1))
        w3 = ~(w0 | w1 | w2)
        row_idx = lane + i * LANES
        for a, w in enumerate((w0, w1, w2, w3)):
            col_idx = jnp.full((LANES,), a, jnp.int32)
            plsc.store_scatter(oht_v, [row_idx, col_idx],
                               jnp.where(w, one, zero))

    c0.wait()
    plsc.parallel_loop(0, CHUNKS_PER_W // 2, 1, unroll=4)(chunk)
    c1.wait()
    plsc.parallel_loop(CHUNKS_PER_W // 2, CHUNKS_PER_W, 1, unroll=4)(chunk)
    pltpu.sync_copy(oht_v, out_hbm.at[pl.ds(base, ROWS_PER_W)])


def kernel(embedded_words, action_vectors):
    return _sc_onehot(_sims_transposed(embedded_words, action_vectors))
